# Initial kernel scaffold; baseline (speedup 1.0000x reference)
import functools

import jax
import jax.numpy as jnp
from jax import lax
from jax.experimental import pallas as pl
from jax.experimental.pallas import tpu as pltpu
from jax.experimental.pallas import tpu_sc as plsc

_B, _N, _OUT = 4, 2048, 13
_F32 = jnp.float32
_INTERPRET = False


def _fps(px, py, pz, npoint):
    b, n = px.shape

    def body(px_ref, py_ref, pz_ref, ox_ref, oy_ref, oz_ref):
        X = px_ref[...]
        Y = py_ref[...]
        Z = pz_ref[...]
        ii_n = lax.broadcasted_iota(jnp.int32, (b, n), 1)
        ii_s = lax.broadcasted_iota(jnp.int32, (b, npoint), 1)

        def step(i, st):
            dist, far, ax, ay, az = st
            m = ii_n == far
            cx = jnp.sum(jnp.where(m, X, 0.0), axis=1, keepdims=True)
            cy = jnp.sum(jnp.where(m, Y, 0.0), axis=1, keepdims=True)
            cz = jnp.sum(jnp.where(m, Z, 0.0), axis=1, keepdims=True)
            sel = ii_s == i
            ax = jnp.where(sel, cx, ax)
            ay = jnp.where(sel, cy, ay)
            az = jnp.where(sel, cz, az)
            d = (X - cx) ** 2 + (Y - cy) ** 2 + (Z - cz) ** 2
            dist = jnp.minimum(dist, d)
            mx = jnp.max(dist, axis=1, keepdims=True)
            far = jnp.min(jnp.where(dist == mx, ii_n, n), axis=1, keepdims=True)
            return (dist, far, ax, ay, az)

        dist0 = jnp.full((b, n), 1e10, _F32)
        far0 = jnp.zeros((b, 1), jnp.int32)
        z0 = jnp.zeros((b, npoint), _F32)
        _, _, ax, ay, az = lax.fori_loop(0, npoint, step, (dist0, far0, z0, z0, z0))
        ox_ref[...] = ax
        oy_ref[...] = ay
        oz_ref[...] = az

    outs = pl.pallas_call(
        body,
        out_shape=[jax.ShapeDtypeStruct((b, npoint), _F32)] * 3,
        interpret=_INTERPRET,
    )(px, py, pz)
    return outs


def _ball_query(s_triple, p_triple, radii, ks):
    sx, sy, sz = s_triple
    px, py, pz = p_triple
    b, s = sx.shape
    n = px.shape[1]
    sx3 = sx[:, :, None]
    sy3 = sy[:, :, None]
    sz3 = sz[:, :, None]
    px3 = px[:, None, :]
    py3 = py[:, None, :]
    pz3 = pz[:, None, :]

    def body(sx_ref, sy_ref, sz_ref, px_ref, py_ref, pz_ref, *o_refs):
        bi = pl.program_id(0)
        A = sx_ref[...].reshape(s, 1)
        Bc = sy_ref[...].reshape(s, 1)
        C = sz_ref[...].reshape(s, 1)
        X = px_ref[...].reshape(1, n)
        Y = py_ref[...].reshape(1, n)
        Z = pz_ref[...].reshape(1, n)
        s2 = A * A + Bc * Bc + C * C
        x2 = X * X + Y * Y + Z * Z
        cross = A * X + Bc * Y + C * Z
        d2 = (s2 + x2) - 2.0 * cross
        ii_n = lax.broadcasted_iota(jnp.int32, (s, n), 1)
        for br, (r, k) in enumerate(zip(radii, ks)):
            mask0 = d2 <= _F32(r * r)
            ii_k = lax.broadcasted_iota(jnp.int32, (s, k), 1)

            def step(kk, st):
                mask, acc, idx0 = st
                cand = jnp.where(mask, ii_n, n)
                idxk = jnp.min(cand, axis=1, keepdims=True)
                idx0 = jnp.where(kk == 0, idxk, idx0)
                idxk = jnp.where(idxk >= n, idx0, idxk)
                mask = mask & (ii_n != idxk)
                acc = jnp.where(ii_k == kk, idxk, acc)
                return (mask, acc, idx0)

            acc0 = jnp.zeros((s, k), jnp.int32)
            idx00 = jnp.zeros((s, 1), jnp.int32)
            _, acc, _ = lax.fori_loop(0, k, step, (mask0, acc0, idx00))
            o_refs[br][...] = (acc + bi * n).reshape(1, s, k)

    outs = pl.pallas_call(
        body,
        grid=(b,),
        in_specs=[
            pl.BlockSpec((1, s, 1), lambda i: (i, 0, 0)),
            pl.BlockSpec((1, s, 1), lambda i: (i, 0, 0)),
            pl.BlockSpec((1, s, 1), lambda i: (i, 0, 0)),
            pl.BlockSpec((1, 1, n), lambda i: (i, 0, 0)),
            pl.BlockSpec((1, 1, n), lambda i: (i, 0, 0)),
            pl.BlockSpec((1, 1, n), lambda i: (i, 0, 0)),
        ],
        out_specs=[pl.BlockSpec((1, s, kk), lambda i: (i, 0, 0)) for kk in ks],
        out_shape=[jax.ShapeDtypeStruct((b, s, kk), jnp.int32) for kk in ks],
        interpret=_INTERPRET,
    )(sx3, sy3, sz3, px3, py3, pz3)
    return outs


def _sc_gather(table, gidx, d):
    r_total = gidx.shape[0]
    nw = 32
    rows_w = r_total // nw
    ch = 128
    nch = rows_w // ch
    idx2d = gidx.reshape(r_total // ch, ch)
    mesh = plsc.VectorSubcoreMesh(core_axis_name="c", subcore_axis_name="s")

    @functools.partial(
        pl.kernel,
        mesh=mesh,
        out_type=jax.ShapeDtypeStruct((r_total, d), _F32),
        scratch_types=[
            pltpu.VMEM((nch, ch), jnp.int32),
            pltpu.VMEM((ch, d), _F32),
            pltpu.VMEM((ch, d), _F32),
            pltpu.SemaphoreType.DMA,
            pltpu.SemaphoreType.DMA,
        ],
    )
    def k(table_hbm, idx_hbm, out_hbm, idx_v, buf0, buf1, sem0, sem1):
        ci = lax.axis_index("c")
        si = lax.axis_index("s")
        wid = si * 2 + ci
        pltpu.sync_copy(idx_hbm.at[pl.ds(wid * nch, nch)], idx_v)
        base = wid * rows_w

        def body(i, _):
            c0 = i * 2
            c1 = i * 2 + 1
            a0 = pltpu.async_copy(table_hbm.at[idx_v.at[c0]], buf0, sem0)
            a1 = pltpu.async_copy(table_hbm.at[idx_v.at[c1]], buf1, sem1)
            a0.wait()
            pltpu.sync_copy(buf0, out_hbm.at[pl.ds(base + c0 * ch, ch)])
            a1.wait()
            pltpu.sync_copy(buf1, out_hbm.at[pl.ds(base + c1 * ch, ch)])
            return 0

        lax.fori_loop(0, nch // 2, body, 0)

    return k(table, idx2d)


def _layer(x, w, bvec, scale=None, shift=None, c=None, wc=None,
           want_stats=True, rb=2048):
    R, cin = x.shape
    cout = w.shape[1]
    if R % rb != 0 or R < rb:
        rb = R
    nb = R // rb
    affine = scale is not None
    has_c = c is not None

    def body(*refs):
        it = iter(refs)
        x_ref = next(it)
        w_ref = next(it)
        b_ref = next(it)
        sc_ref = next(it) if affine else None
        sh_ref = next(it) if affine else None
        c_ref = next(it) if has_c else None
        wc_ref = next(it) if has_c else None
        o_ref = next(it)
        s1_ref = next(it) if want_stats else None
        s2_ref = next(it) if want_stats else None
        xx = x_ref[...]
        if affine:
            xx = jnp.maximum(xx * sc_ref[...] + sh_ref[...], 0.0)
        z = jnp.dot(xx, w_ref[...], preferred_element_type=_F32,
                    precision=lax.Precision.HIGHEST)
        if has_c:
            z = z - jnp.dot(c_ref[...], wc_ref[...], preferred_element_type=_F32,
                            precision=lax.Precision.HIGHEST)
        z = z + b_ref[...]
        o_ref[...] = z
        if want_stats:
            p1 = jnp.sum(z, axis=0, keepdims=True)
            p2 = jnp.sum(z * z, axis=0, keepdims=True)

            @pl.when(pl.program_id(0) == 0)
            def _():
                s1_ref[...] = p1
                s2_ref[...] = p2

            @pl.when(pl.program_id(0) > 0)
            def _():
                s1_ref[...] += p1
                s2_ref[...] += p2

    in_specs = [pl.BlockSpec((rb, cin), lambda i: (i, 0)),
                pl.BlockSpec(w.shape, lambda i: (0, 0)),
                pl.BlockSpec((1, cout), lambda i: (0, 0))]
    args = [x, w, bvec.reshape(1, cout)]
    if affine:
        in_specs += [pl.BlockSpec((1, cin), lambda i: (0, 0)),
                     pl.BlockSpec((1, cin), lambda i: (0, 0))]
        args += [scale.reshape(1, cin), shift.reshape(1, cin)]
    if has_c:
        cdim = c.shape[1]
        in_specs += [pl.BlockSpec((rb, cdim), lambda i: (i, 0)),
                     pl.BlockSpec(wc.shape, lambda i: (0, 0))]
        args += [c, wc]
    out_specs = [pl.BlockSpec((rb, cout), lambda i: (i, 0))]
    out_shape = [jax.ShapeDtypeStruct((R, cout), _F32)]
    if want_stats:
        out_specs += [pl.BlockSpec((1, cout), lambda i: (0, 0))] * 2
        out_shape += [jax.ShapeDtypeStruct((1, cout), _F32)] * 2
    outs = pl.pallas_call(
        body,
        grid=(nb,),
        in_specs=in_specs,
        out_specs=out_specs,
        out_shape=out_shape,
        interpret=_INTERPRET,
    )(*args)
    if want_stats:
        return outs[0], outs[1], outs[2]
    return outs[0]


def _bn(s1, s2, nrows, g, be):
    mean = s1 / nrows
    var = s2 / nrows - mean * mean
    inv = g.reshape(1, -1) / jnp.sqrt(var + 1e-5)
    return inv, be.reshape(1, -1) - mean * inv


def _maxpool(z3, scale, shift, sb=64):
    bs, k, c = z3.shape
    if bs % sb != 0 or bs < sb:
        sb = bs
    nb = bs // sb

    def body(x_ref, sc_ref, sh_ref, o_ref):
        x = x_ref[...]
        h = jnp.maximum(x * sc_ref[...] + sh_ref[...], 0.0)
        o_ref[...] = jnp.max(h, axis=1)

    return pl.pallas_call(
        body,
        grid=(nb,),
        in_specs=[pl.BlockSpec((sb, k, c), lambda i: (i, 0, 0)),
                  pl.BlockSpec((1, 1, c), lambda i: (0, 0, 0)),
                  pl.BlockSpec((1, 1, c), lambda i: (0, 0, 0))],
        out_specs=pl.BlockSpec((sb, c), lambda i: (i, 0)),
        out_shape=jax.ShapeDtypeStruct((bs, c), _F32),
        interpret=_INTERPRET,
    )(z3, scale.reshape(1, 1, c), shift.reshape(1, 1, c))


def _fp_fused(s_triple, d_triple, p1, z2, scale2, shift2, w1a, w1b, b1):
    sx, sy, sz = s_triple
    dx, dy, dz = d_triple
    b, n1 = sx.shape
    n2 = dx.shape[1]
    c1 = p1.shape[-1]
    c2 = z2.shape[-1]
    co = w1a.shape[1]
    sx3 = sx[:, :, None]
    sy3 = sy[:, :, None]
    sz3 = sz[:, :, None]
    dx3 = dx[:, None, :]
    dy3 = dy[:, None, :]
    dz3 = dz[:, None, :]
    z23 = z2.reshape(b, n2, c2)

    def body(sx_ref, sy_ref, sz_ref, dx_ref, dy_ref, dz_ref, p1_ref, z2_ref,
             sc_ref, sh_ref, wa_ref, wb_ref, b_ref, o_ref, s1_ref, s2_ref):
        A = sx_ref[...].reshape(n1, 1)
        Bc = sy_ref[...].reshape(n1, 1)
        C = sz_ref[...].reshape(n1, 1)
        X = dx_ref[...].reshape(1, n2)
        Y = dy_ref[...].reshape(1, n2)
        Z = dz_ref[...].reshape(1, n2)
        s2v = A * A + Bc * Bc + C * C
        x2v = X * X + Y * Y + Z * Z
        cross = A * X + Bc * Y + C * Z
        d2 = (s2v + x2v) - 2.0 * cross
        ii = lax.broadcasted_iota(jnp.int32, (n1, n2), 1)
        d = d2
        wm = jnp.zeros((n1, n2), _F32)
        recips = []
        ixs = []
        for _ in range(3):
            mn = jnp.min(d, axis=1, keepdims=True)
            ix = jnp.min(jnp.where(d == mn, ii, n2), axis=1, keepdims=True)
            d = jnp.where(ii == ix, _F32(3e38), d)
            recips.append(1.0 / (mn + 1e-8))
            ixs.append(ix)
        wsum = recips[0] + recips[1] + recips[2]
        for rec, ix in zip(recips, ixs):
            wm = wm + jnp.where(ii == ix, rec / wsum, 0.0)
        h2 = jnp.maximum(z2_ref[...].reshape(n2, c2) * sc_ref[...]
                         + sh_ref[...], 0.0)
        t = jnp.dot(h2, wb_ref[...], preferred_element_type=_F32,
                    precision=lax.Precision.HIGHEST)
        interp = jnp.dot(wm, t, preferred_element_type=_F32,
                         precision=lax.Precision.HIGHEST)
        z = jnp.dot(p1_ref[...].reshape(n1, c1), wa_ref[...],
                    preferred_element_type=_F32,
                    precision=lax.Precision.HIGHEST) + interp + b_ref[...]
        o_ref[...] = z.reshape(1, n1, co)
        p1s = jnp.sum(z, axis=0, keepdims=True)
        p2s = jnp.sum(z * z, axis=0, keepdims=True)

        @pl.when(pl.program_id(0) == 0)
        def _():
            s1_ref[...] = p1s
            s2_ref[...] = p2s

        @pl.when(pl.program_id(0) > 0)
        def _():
            s1_ref[...] += p1s
            s2_ref[...] += p2s

    outs = pl.pallas_call(
        body,
        grid=(b,),
        in_specs=[
            pl.BlockSpec((1, n1, 1), lambda i: (i, 0, 0)),
            pl.BlockSpec((1, n1, 1), lambda i: (i, 0, 0)),
            pl.BlockSpec((1, n1, 1), lambda i: (i, 0, 0)),
            pl.BlockSpec((1, 1, n2), lambda i: (i, 0, 0)),
            pl.BlockSpec((1, 1, n2), lambda i: (i, 0, 0)),
            pl.BlockSpec((1, 1, n2), lambda i: (i, 0, 0)),
            pl.BlockSpec((1, n1, c1), lambda i: (i, 0, 0)),
            pl.BlockSpec((1, n2, c2), lambda i: (i, 0, 0)),
            pl.BlockSpec((1, c2), lambda i: (0, 0)),
            pl.BlockSpec((1, c2), lambda i: (0, 0)),
            pl.BlockSpec(w1a.shape, lambda i: (0, 0)),
            pl.BlockSpec(w1b.shape, lambda i: (0, 0)),
            pl.BlockSpec((1, co), lambda i: (0, 0)),
        ],
        out_specs=[pl.BlockSpec((1, n1, co), lambda i: (i, 0, 0)),
                   pl.BlockSpec((1, co), lambda i: (0, 0)),
                   pl.BlockSpec((1, co), lambda i: (0, 0))],
        out_shape=[jax.ShapeDtypeStruct((b, n1, co), _F32),
                   jax.ShapeDtypeStruct((1, co), _F32),
                   jax.ShapeDtypeStruct((1, co), _F32)],
        interpret=_INTERPRET,
    )(sx3, sy3, sz3, dx3, dy3, dz3, p1.reshape(b, n1, c1), z23,
      scale2.reshape(1, c2), shift2.reshape(1, c2), w1a, w1b,
      b1.reshape(1, co))
    return outs[0].reshape(b * n1, co), outs[1], outs[2]


def _sa_branch(gathered, c_rep, layers, k, bs):
    R = gathered.shape[0]
    pad_in = gathered.shape[1]
    l0 = layers[0]
    cin = l0['W'].shape[0]
    w1p = jnp.concatenate(
        [l0['W'], jnp.zeros((pad_in - cin, l0['W'].shape[1]), _F32)], axis=0)
    wc = l0['W'][cin - 3:cin]
    z, s1, s2 = _layer(gathered, w1p, l0['b'], c=c_rep, wc=wc)
    sc, sh = _bn(s1, s2, R, l0['g'], l0['be'])
    for l in layers[1:]:
        z, s1n, s2n = _layer(z, l['W'], l['b'], scale=sc, shift=sh)
        sc, sh = _bn(s1n, s2n, R, l['g'], l['be'])
    cz = z.shape[1]
    return _maxpool(z.reshape(bs, k, cz), sc, sh)


def _mlp_chain(x, layers, first_raw=True, scale=None, shift=None):
    R = x.shape[0]
    z = x
    for li, l in enumerate(layers):
        if li == 0 and first_raw:
            z, s1, s2 = _layer(z, l['W'], l['b'])
        else:
            z, s1, s2 = _layer(z, l['W'], l['b'], scale=scale, shift=shift)
        scale, shift = _bn(s1, s2, R, l['g'], l['be'])
    return z, scale, shift


def kernel(xyz, params):
    b, _, n = xyz.shape
    l0 = jnp.transpose(xyz, (0, 2, 1))
    px, py, pz = xyz[:, 0, :], xyz[:, 1, :], xyz[:, 2, :]

    s1n = 512
    ax, ay, az = _fps(px, py, pz, s1n)
    gidx1 = _ball_query((ax, ay, az), (px, py, pz),
                        [0.1, 0.2, 0.4], [32, 64, 128])
    tab1 = jnp.concatenate([l0, l0, jnp.zeros((b, n, 10), _F32)],
                           axis=-1).reshape(b * n, 16)
    new_xyz1 = jnp.stack([ax, ay, az], axis=-1)
    outs1 = []
    for bi, (k, layers) in enumerate(zip([32, 64, 128], params['sa1'])):
        g = _sc_gather(tab1, gidx1[bi].reshape(-1), 16)
        crep = jnp.broadcast_to(new_xyz1[:, :, None, :],
                                (b, s1n, k, 3)).reshape(-1, 3)
        outs1.append(_sa_branch(g, crep, layers, k, b * s1n))
    l1_points = jnp.concatenate(outs1, axis=-1)

    s2n = 128
    bx, by, bz = _fps(ax, ay, az, s2n)
    gidx2 = _ball_query((bx, by, bz), (ax, ay, az), [0.4, 0.8], [64, 128])
    tab2 = jnp.concatenate(
        [l1_points.reshape(b, s1n, 320), new_xyz1,
         jnp.zeros((b, s1n, 13), _F32)], axis=-1).reshape(b * s1n, 336)
    new_xyz2 = jnp.stack([bx, by, bz], axis=-1)
    outs2 = []
    for bi, (k, layers) in enumerate(zip([64, 128], params['sa2'])):
        g = _sc_gather(tab2, gidx2[bi].reshape(-1), 336)
        crep = jnp.broadcast_to(new_xyz2[:, :, None, :],
                                (b, s2n, k, 3)).reshape(-1, 3)
        outs2.append(_sa_branch(g, crep, layers, k, b * s2n))
    l2_points = jnp.concatenate(outs2, axis=-1)

    x3 = jnp.concatenate([new_xyz2.reshape(b * s2n, 3), l2_points], axis=-1)
    z3, sc3, sh3 = _mlp_chain(x3, params['sa3'])
    l3_points = _maxpool(z3.reshape(b, s2n, z3.shape[1]), sc3, sh3)

    rep3 = jnp.broadcast_to(l3_points[:, None, :],
                            (b, s2n, 1024)).reshape(b * s2n, 1024)
    xfp3 = jnp.concatenate([l2_points, rep3], axis=-1)
    zfp3, scf3, shf3 = _mlp_chain(xfp3, params['fp3'])

    wfp2 = params['fp2'][0]['W']
    z, s1, s2 = _fp_fused((ax, ay, az), (bx, by, bz), l1_points, zfp3,
                          scf3, shf3, wfp2[:320], wfp2[320:],
                          params['fp2'][0]['b'])
    scale, shift = _bn(s1, s2, b * s1n, params['fp2'][0]['g'],
                       params['fp2'][0]['be'])
    zfp2, scf2, shf2 = _mlp_chain(z, params['fp2'][1:], first_raw=False,
                                  scale=scale, shift=shift)

    p1_0 = jnp.concatenate([l0, l0], axis=-1).reshape(b * n, 6)
    wfp1 = params['fp1'][0]['W']
    z, s1, s2 = _fp_fused((px, py, pz), (ax, ay, az), p1_0, zfp2,
                          scf2, shf2, wfp1[:6], wfp1[6:],
                          params['fp1'][0]['b'])
    scale, shift = _bn(s1, s2, b * n, params['fp1'][0]['g'],
                       params['fp1'][0]['be'])
    zfp1, scf1, shf1 = _mlp_chain(z, params['fp1'][1:], first_raw=False,
                                  scale=scale, shift=shift)

    h1 = params['head1']
    zh, s1, s2 = _layer(zfp1, h1['W'], h1['b'], scale=scf1, shift=shf1)
    sch, shh = _bn(s1, s2, b * n, h1['g'], h1['be'])
    h2 = params['head2']
    out = _layer(zh, h2['W'], h2['b'], scale=sch, shift=shh,
                 want_stats=False)
    return jnp.transpose(out.reshape(b, n, _OUT), (0, 2, 1))


# speedup vs baseline: 7.9033x; 7.9033x over previous
import functools

import jax
import jax.numpy as jnp
from jax import lax
from jax.experimental import pallas as pl
from jax.experimental.pallas import tpu as pltpu
from jax.experimental.pallas import tpu_sc as plsc

_B, _N, _OUT = 4, 2048, 13
_F32 = jnp.float32


def _fps(px, py, pz, npoint):
    b, n = px.shape

    def body(px_ref, py_ref, pz_ref, ox_ref, oy_ref, oz_ref):
        X = px_ref[...]
        Y = py_ref[...]
        Z = pz_ref[...]
        ii_n = lax.broadcasted_iota(jnp.int32, (b, n), 1)
        ii_s = lax.broadcasted_iota(jnp.int32, (b, npoint), 1)

        def step(i, st):
            dist, far, ax, ay, az = st
            m = ii_n == far
            cx = jnp.sum(jnp.where(m, X, 0.0), axis=1, keepdims=True)
            cy = jnp.sum(jnp.where(m, Y, 0.0), axis=1, keepdims=True)
            cz = jnp.sum(jnp.where(m, Z, 0.0), axis=1, keepdims=True)
            sel = ii_s == i
            ax = jnp.where(sel, cx, ax)
            ay = jnp.where(sel, cy, ay)
            az = jnp.where(sel, cz, az)
            d = (X - cx) ** 2 + (Y - cy) ** 2 + (Z - cz) ** 2
            dist = jnp.minimum(dist, d)
            mx = jnp.max(dist, axis=1, keepdims=True)
            far = jnp.min(jnp.where(dist == mx, ii_n, n), axis=1, keepdims=True)
            return (dist, far, ax, ay, az)

        dist0 = jnp.full((b, n), 1e10, _F32)
        far0 = jnp.zeros((b, 1), jnp.int32)
        z0 = jnp.zeros((b, npoint), _F32)
        _, _, ax, ay, az = lax.fori_loop(0, npoint, step, (dist0, far0, z0, z0, z0))
        ox_ref[...] = ax
        oy_ref[...] = ay
        oz_ref[...] = az

    outs = pl.pallas_call(
        body,
        out_shape=[jax.ShapeDtypeStruct((b, npoint), _F32)] * 3,
    )(px, py, pz)
    return outs


def _ball_query(s_triple, p_triple, radii, ks):
    sx, sy, sz = s_triple
    px, py, pz = p_triple
    b, s = sx.shape
    n = px.shape[1]
    sx3 = sx[:, :, None]
    sy3 = sy[:, :, None]
    sz3 = sz[:, :, None]
    px3 = px[:, None, :]
    py3 = py[:, None, :]
    pz3 = pz[:, None, :]

    def body(sx_ref, sy_ref, sz_ref, px_ref, py_ref, pz_ref, *rest):
        o_refs = rest[:len(ks)]
        d2w_ref, acc_ref = rest[len(ks):]
        bi = pl.program_id(0)
        A = sx_ref[...].reshape(s, 1)
        Bc = sy_ref[...].reshape(s, 1)
        C = sz_ref[...].reshape(s, 1)
        X = px_ref[...].reshape(1, n)
        Y = py_ref[...].reshape(1, n)
        Z = pz_ref[...].reshape(1, n)
        s2 = A * A + Bc * Bc + C * C
        x2 = X * X + Y * Y + Z * Z
        S3 = jnp.concatenate([A, Bc, C], axis=1).astype(jnp.bfloat16)
        P3 = jnp.concatenate([X, Y, Z], axis=0).astype(jnp.bfloat16)
        cross = jnp.dot(S3, P3, preferred_element_type=_F32)
        d2 = (s2 + x2) - 2.0 * cross
        ii_n = lax.broadcasted_iota(jnp.int32, (s, n), 1)
        for br, (r, k) in enumerate(zip(radii, ks)):
            d2w_ref[...] = d2
            ii_k = lax.broadcasted_iota(jnp.int32, (s, k), 1)
            r2 = _F32(r * r)

            def step(kk, idx0):
                dw = d2w_ref[...]
                cand = jnp.where(dw <= r2, ii_n, n)
                idxk = jnp.min(cand, axis=1, keepdims=True)
                idx0 = jnp.where(kk == 0, idxk, idx0)
                idxp = jnp.where(idxk >= n, idx0, idxk)
                d2w_ref[...] = jnp.where(ii_n == idxk, _F32(3e38), dw)
                acc_ref[:, :k] = jnp.where(ii_k == kk, idxp, acc_ref[:, :k])
                return idx0

            lax.fori_loop(0, k, step, jnp.zeros((s, 1), jnp.int32))
            o_refs[br][...] = (jnp.minimum(acc_ref[:, :k], n - 1)
                               + bi * n).reshape(1, s, k)

    outs = pl.pallas_call(
        body,
        grid=(b,),
        in_specs=[
            pl.BlockSpec((1, s, 1), lambda i: (i, 0, 0)),
            pl.BlockSpec((1, s, 1), lambda i: (i, 0, 0)),
            pl.BlockSpec((1, s, 1), lambda i: (i, 0, 0)),
            pl.BlockSpec((1, 1, n), lambda i: (i, 0, 0)),
            pl.BlockSpec((1, 1, n), lambda i: (i, 0, 0)),
            pl.BlockSpec((1, 1, n), lambda i: (i, 0, 0)),
        ],
        out_specs=[pl.BlockSpec((1, s, kk), lambda i: (i, 0, 0)) for kk in ks],
        out_shape=[jax.ShapeDtypeStruct((b, s, kk), jnp.int32) for kk in ks],
        scratch_shapes=[pltpu.VMEM((s, n), _F32),
                        pltpu.VMEM((s, max(ks)), jnp.int32)],
    )(sx3, sy3, sz3, px3, py3, pz3)
    return outs


def _sc_gather(table, gidx, d):
    r_total = gidx.shape[0]
    nw = 32
    rows_w = r_total // nw
    ch = 128
    nch = rows_w // ch
    idx2d = gidx.reshape(r_total // ch, ch)
    mesh = plsc.VectorSubcoreMesh(core_axis_name="c", subcore_axis_name="s")

    @functools.partial(
        pl.kernel,
        mesh=mesh,
        out_type=jax.ShapeDtypeStruct((r_total, d), _F32),
        scratch_types=[
            pltpu.VMEM((nch, ch), jnp.int32),
            pltpu.VMEM((ch, d), _F32),
            pltpu.VMEM((ch, d), _F32),
            pltpu.SemaphoreType.DMA,
            pltpu.SemaphoreType.DMA,
        ],
    )
    def k(table_hbm, idx_hbm, out_hbm, idx_v, buf0, buf1, sem0, sem1):
        ci = lax.axis_index("c")
        si = lax.axis_index("s")
        wid = si * 2 + ci
        pltpu.sync_copy(idx_hbm.at[pl.ds(wid * nch, nch)], idx_v)
        base = wid * rows_w

        def body(i, _):
            c0 = i * 2
            c1 = i * 2 + 1
            a0 = pltpu.async_copy(table_hbm.at[idx_v.at[c0]], buf0, sem0)
            a1 = pltpu.async_copy(table_hbm.at[idx_v.at[c1]], buf1, sem1)
            a0.wait()
            pltpu.sync_copy(buf0, out_hbm.at[pl.ds(base + c0 * ch, ch)])
            a1.wait()
            pltpu.sync_copy(buf1, out_hbm.at[pl.ds(base + c1 * ch, ch)])
            return 0

        lax.fori_loop(0, nch // 2, body, 0)

    return k(table, idx2d)


def _affine(xx, a_ref):
    m = a_ref[0:1, :]
    sq = a_ref[1:2, :]
    g = a_ref[2:3, :]
    be = a_ref[3:4, :]
    return jnp.maximum((xx - m) / sq * g + be, 0.0)


def _layer(x, w, bvec, aff=None, c=None, coff=None,
           want_stats=True, rb=2048):
    R, cin = x.shape
    cout = w.shape[1]
    if R % rb != 0 or R < rb:
        rb = R
    nb = R // rb
    affine = aff is not None
    has_c = c is not None

    def body(*refs):
        it = iter(refs)
        x_ref = next(it)
        w_ref = next(it)
        b_ref = next(it)
        a_ref = next(it) if affine else None
        c_ref = next(it) if has_c else None
        o_ref = next(it)
        s1_ref = next(it) if want_stats else None
        s2_ref = next(it) if want_stats else None
        xx = x_ref[...]
        if has_c:
            cc = c_ref[...]
            cpad = jnp.concatenate(
                [jnp.zeros((rb, coff), _F32), cc,
                 jnp.zeros((rb, cin - coff - 3), _F32)], axis=1)
            xx = xx - cpad
        if affine:
            xx = _affine(xx, a_ref)
        z = jnp.dot(xx.astype(jnp.bfloat16), w_ref[...].astype(jnp.bfloat16),
                    preferred_element_type=_F32)
        z = z + b_ref[...]
        o_ref[...] = z
        if want_stats:
            mu = jnp.sum(z, axis=0, keepdims=True) / rb
            dz = z - mu
            s1_ref[...] = mu.reshape(1, 1, cout)
            s2_ref[...] = jnp.sum(dz * dz, axis=0, keepdims=True).reshape(1, 1, cout)

    in_specs = [pl.BlockSpec((rb, cin), lambda i: (i, 0)),
                pl.BlockSpec(w.shape, lambda i: (0, 0)),
                pl.BlockSpec((1, cout), lambda i: (0, 0))]
    args = [x, w, bvec.reshape(1, cout)]
    if affine:
        in_specs += [pl.BlockSpec((4, cin), lambda i: (0, 0))]
        args += [aff]
    if has_c:
        cdim = c.shape[1]
        in_specs += [pl.BlockSpec((rb, cdim), lambda i: (i, 0))]
        args += [c]
    out_specs = [pl.BlockSpec((rb, cout), lambda i: (i, 0))]
    out_shape = [jax.ShapeDtypeStruct((R, cout), _F32)]
    if want_stats:
        out_specs += [pl.BlockSpec((1, 1, cout), lambda i: (i, 0, 0))] * 2
        out_shape += [jax.ShapeDtypeStruct((nb, 1, cout), _F32)] * 2
    outs = pl.pallas_call(
        body,
        grid=(nb,),
        in_specs=in_specs,
        out_specs=out_specs,
        out_shape=out_shape,
    )(*args)
    if want_stats:
        return outs[0], outs[1].reshape(nb, cout), outs[2].reshape(nb, cout)
    return outs[0]


def _bn(mu, m2, nrows, g, be):
    nb = mu.shape[0]
    rbk = nrows // nb
    mean = jnp.mean(mu, axis=0, keepdims=True)
    var = (jnp.sum(m2, axis=0, keepdims=True)
           + rbk * jnp.sum((mu - mean) ** 2, axis=0, keepdims=True)) / nrows
    sq = jnp.sqrt(var + 1e-5)
    return jnp.concatenate(
        [mean, sq, g.reshape(1, -1), be.reshape(1, -1)], axis=0)


def _maxpool(z3, aff, sb=64):
    bs, k, c = z3.shape
    if bs % sb != 0 or bs < sb:
        sb = bs
    nb = bs // sb

    def body(x_ref, a_ref, o_ref):
        x = x_ref[...]
        a = a_ref[...]
        h = jnp.maximum((x - a[0:1, 0:1, :]) / a[0:1, 1:2, :]
                        * a[0:1, 2:3, :] + a[0:1, 3:4, :], 0.0)
        o_ref[...] = jnp.max(h, axis=1)

    return pl.pallas_call(
        body,
        grid=(nb,),
        in_specs=[pl.BlockSpec((sb, k, c), lambda i: (i, 0, 0)),
                  pl.BlockSpec((1, 4, c), lambda i: (0, 0, 0))],
        out_specs=pl.BlockSpec((sb, c), lambda i: (i, 0)),
        out_shape=jax.ShapeDtypeStruct((bs, c), _F32),
    )(z3, aff.reshape(1, 4, c))


def _fp_fused(s_triple, d_triple, p1, z2, aff2, w1a, w1b, b1):
    sx, sy, sz = s_triple
    dx, dy, dz = d_triple
    b, n1 = sx.shape
    n2 = dx.shape[1]
    c1 = p1.shape[-1]
    c2 = z2.shape[-1]
    co = w1a.shape[1]
    sx3 = sx[:, :, None]
    sy3 = sy[:, :, None]
    sz3 = sz[:, :, None]
    dx3 = dx[:, None, :]
    dy3 = dy[:, None, :]
    dz3 = dz[:, None, :]
    z23 = z2.reshape(b, n2, c2)

    def body(sx_ref, sy_ref, sz_ref, dx_ref, dy_ref, dz_ref, p1_ref, z2_ref,
             a_ref, wa_ref, wb_ref, b_ref, o_ref, s1_ref, s2_ref):
        A = sx_ref[...].reshape(n1, 1)
        Bc = sy_ref[...].reshape(n1, 1)
        C = sz_ref[...].reshape(n1, 1)
        X = dx_ref[...].reshape(1, n2)
        Y = dy_ref[...].reshape(1, n2)
        Z = dz_ref[...].reshape(1, n2)
        s2v = A * A + Bc * Bc + C * C
        x2v = X * X + Y * Y + Z * Z
        S3 = jnp.concatenate([A, Bc, C], axis=1).astype(jnp.bfloat16)
        P3 = jnp.concatenate([X, Y, Z], axis=0).astype(jnp.bfloat16)
        cross = jnp.dot(S3, P3, preferred_element_type=_F32)
        d2 = (s2v + x2v) - 2.0 * cross
        ii = lax.broadcasted_iota(jnp.int32, (n1, n2), 1)
        d = d2
        wm = jnp.zeros((n1, n2), _F32)
        recips = []
        ixs = []
        for _ in range(3):
            mn = jnp.min(d, axis=1, keepdims=True)
            ix = jnp.min(jnp.where(d == mn, ii, n2), axis=1, keepdims=True)
            d = jnp.where(ii == ix, _F32(3e38), d)
            recips.append(1.0 / (mn + 1e-8))
            ixs.append(ix)
        wsum = recips[0] + recips[1] + recips[2]
        for rec, ix in zip(recips, ixs):
            wm = wm + jnp.where(ii == ix, rec / wsum, 0.0)
        h2 = _affine(z2_ref[...].reshape(n2, c2), a_ref)
        interp = jnp.dot(wm, h2, preferred_element_type=_F32,
                         precision=lax.Precision.HIGHEST)
        z = (jnp.dot(p1_ref[...].reshape(n1, c1).astype(jnp.bfloat16),
                     wa_ref[...].astype(jnp.bfloat16),
                     preferred_element_type=_F32)
             + jnp.dot(interp.astype(jnp.bfloat16),
                       wb_ref[...].astype(jnp.bfloat16),
                       preferred_element_type=_F32)
             + b_ref[...])
        o_ref[...] = z.reshape(1, n1, co)
        mu = jnp.sum(z, axis=0, keepdims=True) / n1
        dz = z - mu
        s1_ref[...] = mu.reshape(1, 1, co)
        s2_ref[...] = jnp.sum(dz * dz, axis=0, keepdims=True).reshape(1, 1, co)

    outs = pl.pallas_call(
        body,
        grid=(b,),
        in_specs=[
            pl.BlockSpec((1, n1, 1), lambda i: (i, 0, 0)),
            pl.BlockSpec((1, n1, 1), lambda i: (i, 0, 0)),
            pl.BlockSpec((1, n1, 1), lambda i: (i, 0, 0)),
            pl.BlockSpec((1, 1, n2), lambda i: (i, 0, 0)),
            pl.BlockSpec((1, 1, n2), lambda i: (i, 0, 0)),
            pl.BlockSpec((1, 1, n2), lambda i: (i, 0, 0)),
            pl.BlockSpec((1, n1, c1), lambda i: (i, 0, 0)),
            pl.BlockSpec((1, n2, c2), lambda i: (i, 0, 0)),
            pl.BlockSpec((4, c2), lambda i: (0, 0)),
            pl.BlockSpec(w1a.shape, lambda i: (0, 0)),
            pl.BlockSpec(w1b.shape, lambda i: (0, 0)),
            pl.BlockSpec((1, co), lambda i: (0, 0)),
        ],
        out_specs=[pl.BlockSpec((1, n1, co), lambda i: (i, 0, 0)),
                   pl.BlockSpec((1, 1, co), lambda i: (i, 0, 0)),
                   pl.BlockSpec((1, 1, co), lambda i: (i, 0, 0))],
        out_shape=[jax.ShapeDtypeStruct((b, n1, co), _F32),
                   jax.ShapeDtypeStruct((b, 1, co), _F32),
                   jax.ShapeDtypeStruct((b, 1, co), _F32)],
    )(sx3, sy3, sz3, dx3, dy3, dz3, p1.reshape(b, n1, c1), z23,
      aff2, w1a, w1b, b1.reshape(1, co))
    return outs[0].reshape(b * n1, co), outs[1].reshape(b, co), outs[2].reshape(b, co)


def _sa_branch(gathered, c_rep, layers, k, bs):
    R = gathered.shape[0]
    pad_in = gathered.shape[1]
    l0 = layers[0]
    cin = l0['W'].shape[0]
    w1p = jnp.concatenate(
        [l0['W'], jnp.zeros((pad_in - cin, l0['W'].shape[1]), _F32)], axis=0)
    z, s1, s2 = _layer(gathered, w1p, l0['b'], c=c_rep, coff=cin - 3)
    aff = _bn(s1, s2, R, l0['g'], l0['be'])
    for l in layers[1:]:
        z, s1n, s2n = _layer(z, l['W'], l['b'], aff=aff)
        aff = _bn(s1n, s2n, R, l['g'], l['be'])
    cz = z.shape[1]
    return _maxpool(z.reshape(bs, k, cz), aff)


def _mlp_chain(x, layers, first_raw=True, aff=None):
    R = x.shape[0]
    z = x
    for li, l in enumerate(layers):
        if li == 0 and first_raw:
            z, s1, s2 = _layer(z, l['W'], l['b'])
        else:
            z, s1, s2 = _layer(z, l['W'], l['b'], aff=aff)
        aff = _bn(s1, s2, R, l['g'], l['be'])
    return z, aff


def kernel(xyz, params):
    b, _, n = xyz.shape
    l0 = jnp.transpose(xyz, (0, 2, 1))
    px, py, pz = xyz[:, 0, :], xyz[:, 1, :], xyz[:, 2, :]

    s1n = 512
    ax, ay, az = _fps(px, py, pz, s1n)
    gidx1 = _ball_query((ax, ay, az), (px, py, pz),
                        [0.1, 0.2, 0.4], [32, 64, 128])
    tab1 = jnp.concatenate([l0, l0, jnp.zeros((b, n, 122), _F32)],
                           axis=-1).reshape(b * n, 128)
    new_xyz1 = jnp.stack([ax, ay, az], axis=-1)
    outs1 = []
    for bi, (k, layers) in enumerate(zip([32, 64, 128], params['sa1'])):
        g = _sc_gather(tab1, gidx1[bi].reshape(-1), 128)
        crep = jnp.broadcast_to(new_xyz1[:, :, None, :],
                                (b, s1n, k, 3)).reshape(-1, 3)
        outs1.append(_sa_branch(g, crep, layers, k, b * s1n))
    l1_points = jnp.concatenate(outs1, axis=-1)

    s2n = 128
    bx, by, bz = _fps(ax, ay, az, s2n)
    gidx2 = _ball_query((bx, by, bz), (ax, ay, az), [0.4, 0.8], [64, 128])
    tab2 = jnp.concatenate(
        [l1_points.reshape(b, s1n, 320), new_xyz1,
         jnp.zeros((b, s1n, 61), _F32)], axis=-1).reshape(b * s1n, 384)
    new_xyz2 = jnp.stack([bx, by, bz], axis=-1)
    outs2 = []
    for bi, (k, layers) in enumerate(zip([64, 128], params['sa2'])):
        g = _sc_gather(tab2, gidx2[bi].reshape(-1), 384)
        crep = jnp.broadcast_to(new_xyz2[:, :, None, :],
                                (b, s2n, k, 3)).reshape(-1, 3)
        outs2.append(_sa_branch(g, crep, layers, k, b * s2n))
    l2_points = jnp.concatenate(outs2, axis=-1)

    x3 = jnp.concatenate([new_xyz2.reshape(b * s2n, 3), l2_points], axis=-1)
    z3, aff3 = _mlp_chain(x3, params['sa3'])
    l3_points = _maxpool(z3.reshape(b, s2n, z3.shape[1]), aff3)

    rep3 = jnp.broadcast_to(l3_points[:, None, :],
                            (b, s2n, 1024)).reshape(b * s2n, 1024)
    xfp3 = jnp.concatenate([l2_points, rep3], axis=-1)
    zfp3, afff3 = _mlp_chain(xfp3, params['fp3'])

    wfp2 = params['fp2'][0]['W']
    z, s1, s2 = _fp_fused((ax, ay, az), (bx, by, bz), l1_points, zfp3,
                          afff3, wfp2[:320], wfp2[320:],
                          params['fp2'][0]['b'])
    aff = _bn(s1, s2, b * s1n, params['fp2'][0]['g'], params['fp2'][0]['be'])
    zfp2, afff2 = _mlp_chain(z, params['fp2'][1:], first_raw=False, aff=aff)

    p1_0 = jnp.concatenate([l0, l0], axis=-1).reshape(b * n, 6)
    wfp1 = params['fp1'][0]['W']
    z, s1, s2 = _fp_fused((px, py, pz), (ax, ay, az), p1_0, zfp2,
                          afff2, wfp1[:6], wfp1[6:],
                          params['fp1'][0]['b'])
    aff = _bn(s1, s2, b * n, params['fp1'][0]['g'], params['fp1'][0]['be'])
    zfp1, afff1 = _mlp_chain(z, params['fp1'][1:], first_raw=False, aff=aff)

    h1 = params['head1']
    zh, s1, s2 = _layer(zfp1, h1['W'], h1['b'], aff=afff1)
    affh = _bn(s1, s2, b * n, h1['g'], h1['be'])
    h2 = params['head2']
    out = _layer(zh, h2['W'], h2['b'], aff=affh, want_stats=False)
    return jnp.transpose(out.reshape(b, n, _OUT), (0, 2, 1))
